# Initial kernel scaffold; baseline (speedup 1.0000x reference)
import jax
import jax.numpy as jnp
from jax.experimental import pallas as pl


def kernel(input, positional, W, P):
    raise NotImplementedError("write your pallas kernel here")


# speedup vs baseline: 5.6637x; 5.6637x over previous
import functools

import jax
import jax.numpy as jnp
from jax import lax
from jax.experimental import pallas as pl
from jax.experimental.pallas import tpu as pltpu
from jax.experimental.pallas import tpu_sc as plsc

B_S = 1024
S_L = 200
H = 128
N = B_S * S_L
NC, NS, L = 2, 16, 16
NW = NC * NS
TPW = N // NW
C = 128
NCHUNK = TPW // C
COLS = H // L

_mesh = plsc.VectorSubcoreMesh(core_axis_name="c", subcore_axis_name="s")


@functools.partial(
    pl.kernel,
    out_type=[
        jax.ShapeDtypeStruct((N, H), jnp.float32),
        jax.ShapeDtypeStruct((NW, NCHUNK, C), jnp.int32),
    ],
    mesh=_mesh,
    scratch_types=[
        pltpu.VMEM((NCHUNK, C), jnp.int32),
        pltpu.VMEM((NCHUNK, C), jnp.int32),
        pltpu.VMEM((NCHUNK, C), jnp.int32),
        pltpu.VMEM((C, H), jnp.float32),
        pltpu.VMEM((C, H), jnp.float32),
        pltpu.SemaphoreType.DMA,
        pltpu.SemaphoreType.DMA,
    ],
)
def _emb(inp_hbm, pos_hbm, w_hbm, p_hbm, out_hbm, mask_hbm,
         tok_v, pos_v, msk_v, wrows, prows, semw, semp):
    wid = lax.axis_index("s") * NC + lax.axis_index("c")
    base = wid * TPW

    pltpu.sync_copy(inp_hbm.at[wid], tok_v)
    pltpu.sync_copy(pos_hbm.at[wid], pos_v)

    ones = jnp.ones((L,), jnp.int32)

    def mask_row(j, _):
        def mask_col(i, _):
            t = tok_v[j, pl.ds(i * L, L)]
            p = pos_v[j, pl.ds(i * L, L)]
            nonpad = jnp.minimum(jnp.abs(t), ones)
            pos_v[j, pl.ds(i * L, L)] = p * nonpad
            msk_v[j, pl.ds(i * L, L)] = ones - nonpad
            return 0
        return lax.fori_loop(0, C // L, mask_col, 0)

    lax.fori_loop(0, NCHUNK, mask_row, 0)
    pltpu.sync_copy(msk_v, mask_hbm.at[wid])

    def chunk(j, _):
        cw = pltpu.async_copy(w_hbm.at[tok_v.at[j]], wrows, semw)
        cp = pltpu.async_copy(p_hbm.at[pos_v.at[j]], prows, semp)
        cw.wait()
        cp.wait()

        def add_row(r, _):
            for c in range(COLS):
                sl = pl.ds(c * L, L)
                wrows[r, sl] = wrows[r, sl] + prows[r, sl]
            return 0

        lax.fori_loop(0, C, add_row, 0)
        pltpu.sync_copy(wrows, out_hbm.at[pl.ds(base + j * C, C)])
        return 0

    lax.fori_loop(0, NCHUNK, chunk, 0)


def kernel(input, positional, W, P):
    inp = input.astype(jnp.int32).reshape(NW, NCHUNK, C)
    pos = positional.astype(jnp.int32).reshape(NW, NCHUNK, C)
    out, mask = _emb(inp, pos, W, P)
    return (out.reshape(B_S, S_L, H),
            mask.reshape(B_S, S_L).astype(bool))


# speedup vs baseline: 11.2265x; 1.9822x over previous
import functools

import jax
import jax.numpy as jnp
from jax import lax
from jax.experimental import pallas as pl
from jax.experimental.pallas import tpu as pltpu
from jax.experimental.pallas import tpu_sc as plsc

B_S = 1024
S_L = 200
H = 128
MAX_LEN = 512
N = B_S * S_L
NC, NS, L = 2, 16, 16
NW = NC * NS
TPW = N // NW
C = 64
NCHUNK = TPW // C
COLS = H // L
NBW = 4
NBP = 2

_mesh = plsc.VectorSubcoreMesh(core_axis_name="c", subcore_axis_name="s")


@functools.partial(
    pl.kernel,
    out_type=[
        jax.ShapeDtypeStruct((N, H), jnp.float32),
        jax.ShapeDtypeStruct((NW, NCHUNK, C), jnp.int32),
    ],
    mesh=_mesh,
    scratch_types=[
        pltpu.VMEM((NCHUNK, C), jnp.int32),
        pltpu.VMEM((NCHUNK, C), jnp.int32),
        pltpu.VMEM((NCHUNK, C), jnp.int32),
        pltpu.VMEM((NBW, C, H), jnp.float32),
        pltpu.VMEM((NBP, C, H), jnp.float32),
        pltpu.VMEM_SHARED((MAX_LEN, H), jnp.float32),
    ] + [pltpu.SemaphoreType.DMA] * (NBW + NBP + NBW),
)
def _emb(inp_hbm, pos_hbm, w_hbm, p_hbm, out_hbm, mask_hbm,
         tok_v, pos_v, msk_v, wrows, prows, p_sh, *sems):
    semw = sems[:NBW]
    semp = sems[NBW:NBW + NBP]
    semo = sems[NBW + NBP:]
    wid = lax.axis_index("s") * NC + lax.axis_index("c")
    sid = lax.axis_index("s")
    base = wid * TPW

    @pl.when(sid == 0)
    def _stage_p():
        pltpu.sync_copy(p_hbm, p_sh)

    pltpu.sync_copy(inp_hbm.at[wid], tok_v)
    pltpu.sync_copy(pos_hbm.at[wid], pos_v)
    plsc.subcore_barrier()

    ones = jnp.ones((L,), jnp.int32)

    def mask_chunk(j):
        def mc(i, _):
            t = tok_v[j, pl.ds(i * L, L)]
            p = pos_v[j, pl.ds(i * L, L)]
            nonpad = jnp.minimum(jnp.abs(t), ones)
            pos_v[j, pl.ds(i * L, L)] = p * nonpad
            msk_v[j, pl.ds(i * L, L)] = ones - nonpad
            return 0
        lax.fori_loop(0, C // L, mc, 0)

    def issue_gathers(j, bw, bp):
        pltpu.async_copy(w_hbm.at[tok_v.at[j]], wrows.at[bw], semw[bw])
        pltpu.async_copy(p_sh.at[pos_v.at[j]], prows.at[bp], semp[bp])

    def wait_slot(sem, bw):
        pltpu.make_async_copy(w_hbm.at[pl.ds(0, C)], wrows.at[bw], sem).wait()

    mask_chunk(0)
    mask_chunk(1)
    issue_gathers(0, 0, 0)
    issue_gathers(1, 1, 1)

    def outer(j0, _):
        for b in range(NBW):
            j = j0 * NBW + b
            bw = b
            bp = b % NBP
            wait_slot(semw[bw], bw)
            wait_slot(semp[bp], bw)

            def add_row(r, _):
                for c in range(COLS):
                    sl = pl.ds(c * L, L)
                    plsc.addupdate(wrows.at[bw, r, sl], prows[bp, r, sl])
                return 0

            lax.fori_loop(0, C, add_row, 0)
            pltpu.async_copy(wrows.at[bw],
                             out_hbm.at[pl.ds(base + j * C, C)], semo[bw])
            jn = j + 2

            @pl.when(jn < NCHUNK)
            def _prefetch():
                mask_chunk(jn)

            @pl.when(j >= 2)
            def _drain_store():
                wait_slot(semo[(b + 2) % NBW], bw)

            @pl.when(jn < NCHUNK)
            def _issue_next():
                issue_gathers(jn, (b + 2) % NBW, bp)
        return 0

    lax.fori_loop(0, NCHUNK // NBW, outer, 0)
    wait_slot(semo[(NCHUNK - 2) % NBW], 0)
    wait_slot(semo[(NCHUNK - 1) % NBW], 1)
    pltpu.sync_copy(msk_v, mask_hbm.at[wid])


def kernel(input, positional, W, P):
    inp = input.astype(jnp.int32).reshape(NW, NCHUNK, C)
    pos = positional.astype(jnp.int32).reshape(NW, NCHUNK, C)
    out, mask = _emb(inp, pos, W, P)
    return (out.reshape(B_S, S_L, H),
            mask.reshape(B_S, S_L).astype(bool))


# speedup vs baseline: 13.3400x; 1.1883x over previous
import functools

import jax
import jax.numpy as jnp
from jax import lax
from jax.experimental import pallas as pl
from jax.experimental.pallas import tpu as pltpu
from jax.experimental.pallas import tpu_sc as plsc

B_S = 1024
S_L = 200
H = 128
MAX_LEN = 512
N = B_S * S_L
NC, NS, L = 2, 16, 16
NW = NC * NS
TPW = N // NW
C = 64
NCHUNK = TPW // C
COLS = H // L
NB = 4
DIST = 3
PROWS = MAX_LEN // NS

_mesh = plsc.VectorSubcoreMesh(core_axis_name="c", subcore_axis_name="s")


@functools.partial(
    pl.kernel,
    out_type=[
        jax.ShapeDtypeStruct((N, H), jnp.float32),
        jax.ShapeDtypeStruct((NW, NCHUNK, C), jnp.int32),
    ],
    mesh=_mesh,
    scratch_types=[
        pltpu.VMEM((NCHUNK, C), jnp.int32),
        pltpu.VMEM((NCHUNK, C), jnp.int32),
        pltpu.VMEM((NCHUNK, C), jnp.int32),
        pltpu.VMEM((NB, C, H), jnp.float32),
        pltpu.VMEM((NB, C, H), jnp.float32),
        pltpu.VMEM_SHARED((MAX_LEN, H), jnp.float32),
    ] + [pltpu.SemaphoreType.DMA] * (3 * NB + 3),
)
def _emb(inp_hbm, pos_hbm, w_hbm, p_hbm, out_hbm, mask_hbm,
         tok_v, pos_v, msk_v, wrows, prows, p_sh, *sems):
    semw = sems[:NB]
    semp = sems[NB:2 * NB]
    semo = sems[2 * NB:3 * NB]
    semt, semq, semm = sems[3 * NB:]
    wid = lax.axis_index("s") * NC + lax.axis_index("c")
    sid = lax.axis_index("s")
    base = wid * TPW

    prow0 = sid * PROWS
    pltpu.async_copy(p_hbm.at[pl.ds(prow0, PROWS)],
                     p_sh.at[pl.ds(prow0, PROWS)], semm)
    pltpu.async_copy(inp_hbm.at[wid], tok_v, semt)
    pltpu.async_copy(pos_hbm.at[wid], pos_v, semq)
    pltpu.make_async_copy(p_hbm.at[pl.ds(0, PROWS)],
                          p_sh.at[pl.ds(0, PROWS)], semm).wait()
    pltpu.make_async_copy(inp_hbm.at[wid], tok_v, semt).wait()
    pltpu.make_async_copy(pos_hbm.at[wid], pos_v, semq).wait()
    plsc.subcore_barrier()

    ones = jnp.ones((L,), jnp.int32)

    def mask_chunk(j):
        def mc(i, _):
            t = tok_v[j, pl.ds(i * L, L)]
            p = pos_v[j, pl.ds(i * L, L)]
            nonpad = jnp.minimum(jnp.abs(t), ones)
            pos_v[j, pl.ds(i * L, L)] = p * nonpad
            msk_v[j, pl.ds(i * L, L)] = ones - nonpad
            return 0
        lax.fori_loop(0, C // L, mc, 0)

    def issue_gathers(j, b):
        pltpu.async_copy(w_hbm.at[tok_v.at[j]], wrows.at[b], semw[b])
        pltpu.async_copy(p_sh.at[pos_v.at[j]], prows.at[b], semp[b])

    def wait_slot(sem, b):
        pltpu.make_async_copy(w_hbm.at[pl.ds(0, C)], wrows.at[b], sem).wait()

    for j in range(DIST):
        mask_chunk(j)
        issue_gathers(j, j)

    def outer(j0, _):
        for b in range(NB):
            j = j0 * NB + b
            wait_slot(semw[b], b)
            wait_slot(semp[b], b)

            def add_row(r, _):
                for c in range(COLS):
                    sl = pl.ds(c * L, L)
                    plsc.addupdate(wrows.at[b, r, sl], prows[b, r, sl])
                return 0

            lax.fori_loop(0, C, add_row, 0)
            pltpu.async_copy(wrows.at[b],
                             out_hbm.at[pl.ds(base + j * C, C)], semo[b])
            jn = j + DIST

            @pl.when(jn < NCHUNK)
            def _prefetch():
                mask_chunk(jn)

            @pl.when(j == NCHUNK - DIST - 1)
            def _store_mask():
                pltpu.async_copy(msk_v, mask_hbm.at[wid], semm)

            @pl.when(j >= NB - DIST)
            def _drain_store():
                wait_slot(semo[(b + DIST) % NB], b)

            @pl.when(jn < NCHUNK)
            def _issue_next():
                issue_gathers(jn, (b + DIST) % NB)
        return 0

    lax.fori_loop(0, NCHUNK // NB, outer, 0)
    wait_slot(semo[(NCHUNK - 1) % NB], 0)
    pltpu.make_async_copy(msk_v, mask_hbm.at[wid], semm).wait()


def kernel(input, positional, W, P):
    inp = input.astype(jnp.int32).reshape(NW, NCHUNK, C)
    pos = positional.astype(jnp.int32).reshape(NW, NCHUNK, C)
    out, mask = _emb(inp, pos, W, P)
    return (out.reshape(B_S, S_L, H),
            mask.reshape(B_S, S_L).astype(bool))
